# baseline (speedup 1.0000x reference)
import functools

import jax
import jax.numpy as jnp
from jax import lax
from jax.experimental import pallas as pl
from jax.experimental.pallas import tpu as pltpu
from jax.experimental.pallas import tpu_sc as plsc

N = 10000
E = 160000
D_IN = 256
D_H = 64
D_OUT = 256
DW = 128

NC = 2
NS = 16
NW = NC * NS
E_PER_W = E // NW
CHUNK = 125
NCHUNK = E_PER_W // CHUNK
SPAN = 640
LAST_SPAN = N - (NS - 1) * SPAN

R = 2000
G = N // R


def _lin0_body(x_ref, w_ref, b_ref, o_ref):
    o = jnp.dot(x_ref[...], w_ref[...], preferred_element_type=jnp.float32)
    o_ref[:, :D_H] = jnp.maximum(o + b_ref[...], 0.0)
    o_ref[:, D_H:] = jnp.zeros((R, DW - D_H), jnp.float32)


def _tc_lin0(x, w, b):
    return pl.pallas_call(
        _lin0_body,
        grid=(G,),
        in_specs=[
            pl.BlockSpec((R, D_IN), lambda i: (i, 0)),
            pl.BlockSpec((D_IN, D_H), lambda i: (0, 0)),
            pl.BlockSpec((1, D_H), lambda i: (0, 0)),
        ],
        out_specs=pl.BlockSpec((R, DW), lambda i: (i, 0)),
        out_shape=jax.ShapeDtypeStruct((N, DW), jnp.float32),
    )(x, w, b)


def _conv_body(acc_ref, degt_ref, h_ref, wn_ref, wr_ref, b_ref, o_ref):
    d = degt_ref[...]
    ds = d[:, 0:1] + d[:, 1:2]
    inv = 1.0 / jnp.maximum(ds, 1.0)
    mean = (acc_ref[0, :, :D_H] + acc_ref[1, :, :D_H]) * inv
    o = (jnp.dot(mean, wn_ref[...], preferred_element_type=jnp.float32)
         + jnp.dot(h_ref[:, :D_H], wr_ref[...],
                   preferred_element_type=jnp.float32))
    o_ref[:, :D_H] = jnp.maximum(o + b_ref[...], 0.0)
    o_ref[:, D_H:] = jnp.zeros((R, DW - D_H), jnp.float32)


def _tc_conv(acc, degt, h, wn, wr, b):
    return pl.pallas_call(
        _conv_body,
        grid=(G,),
        in_specs=[
            pl.BlockSpec((2, R, DW), lambda i: (0, i, 0)),
            pl.BlockSpec((R, 2), lambda i: (i, 0)),
            pl.BlockSpec((R, DW), lambda i: (i, 0)),
            pl.BlockSpec((D_H, D_H), lambda i: (0, 0)),
            pl.BlockSpec((D_H, D_H), lambda i: (0, 0)),
            pl.BlockSpec((1, D_H), lambda i: (0, 0)),
        ],
        out_specs=pl.BlockSpec((R, DW), lambda i: (i, 0)),
        out_shape=jax.ShapeDtypeStruct((N, DW), jnp.float32),
    )(acc, degt, h, wn, wr, b)


def _final_body(acc_ref, degt_ref, h_ref, wn_ref, wr_ref, b_ref,
                lw_ref, lb_ref, o_ref):
    d = degt_ref[...]
    ds = d[:, 0:1] + d[:, 1:2]
    inv = 1.0 / jnp.maximum(ds, 1.0)
    mean = (acc_ref[0, :, :D_H] + acc_ref[1, :, :D_H]) * inv
    x2 = (jnp.dot(mean, wn_ref[...], preferred_element_type=jnp.float32)
          + jnp.dot(h_ref[:, :D_H], wr_ref[...],
                    preferred_element_type=jnp.float32))
    x2 = jnp.maximum(x2 + b_ref[...], 0.0)
    o = jnp.dot(x2, lw_ref[...], preferred_element_type=jnp.float32) + lb_ref[...]
    m = jnp.max(o, axis=1, keepdims=True)
    sh = o - m
    lse = jnp.log(jnp.sum(jnp.exp(sh), axis=1, keepdims=True))
    o_ref[...] = sh - lse


def _tc_final(acc, degt, h, wn, wr, b, lw, lb):
    return pl.pallas_call(
        _final_body,
        grid=(G,),
        in_specs=[
            pl.BlockSpec((2, R, DW), lambda i: (0, i, 0)),
            pl.BlockSpec((R, 2), lambda i: (i, 0)),
            pl.BlockSpec((R, DW), lambda i: (i, 0)),
            pl.BlockSpec((D_H, D_H), lambda i: (0, 0)),
            pl.BlockSpec((D_H, D_H), lambda i: (0, 0)),
            pl.BlockSpec((1, D_H), lambda i: (0, 0)),
            pl.BlockSpec((D_H, D_OUT), lambda i: (0, 0)),
            pl.BlockSpec((1, D_OUT), lambda i: (0, 0)),
        ],
        out_specs=pl.BlockSpec((R, D_OUT), lambda i: (i, 0)),
        out_shape=jax.ShapeDtypeStruct((N, D_OUT), jnp.float32),
    )(acc, degt, h, wn, wr, b, lw, lb)


DEPTH = 2
NQ = NCHUNK // DEPTH


def _spans(sid, fn640, fn400):
    @pl.when(sid < NS - 1)
    def _():
        fn640(pl.multiple_of(sid * SPAN, 8))

    @pl.when(sid == NS - 1)
    def _():
        fn400()


def _stage_between(src, dst, stage, src_off, dst_off, length):
    pltpu.sync_copy(src.at[pl.ds(src_off, length)], stage.at[pl.ds(0, length)])
    pltpu.sync_copy(stage.at[pl.ds(0, length)], dst.at[pl.ds(dst_off, length)])


def _edge_loop(table, idx_s, idx_d, rows, gsems, ssems, acc_sh, deg=None):
    def gather(c, k):
        pltpu.async_copy(table.at[idx_s.at[c]], rows[k], gsems[k])

    def gwait(k):
        pltpu.make_async_copy(table.at[idx_s.at[0]], rows[k], gsems[k]).wait()

    def swait(k):
        pltpu.make_async_copy(table.at[idx_s.at[0]], rows[k], ssems[k]).wait()

    for k in range(DEPTH):
        gather(k, k)

    def step(i, carry):
        for k in range(DEPTH):
            c = DEPTH * i + k
            gwait(k)
            pltpu.async_copy(rows[k], acc_sh.at[idx_d.at[c]], ssems[k],
                             add=True)
            if deg is not None:
                ones_v, deg_sh = deg
                pltpu.sync_copy(ones_v, deg_sh.at[idx_d.at[c]], add=True)

        @pl.when(i < NQ - 1)
        def _():
            for k in range(DEPTH):
                swait(k)
                gather(DEPTH * (i + 1) + k, k)

        return carry

    lax.fori_loop(0, NQ, step, 0)
    for k in range(DEPTH):
        swait(k)


def _sc_body_deg(table, srcr, dstr, z64, zdeg, ones_in,
                 acc_out, deg_out,
                 idx_s, idx_d, r0b, r1b, ones_v, stage1,
                 g0, g1, s0, s1,
                 acc_sh, deg_sh):
    cid = lax.axis_index("c")
    sid = lax.axis_index("s")
    wid = cid * NS + sid
    _spans(sid,
           lambda r0: pltpu.sync_copy(z64, acc_sh.at[pl.ds(r0, SPAN)]),
           lambda: pltpu.sync_copy(z64.at[pl.ds(0, LAST_SPAN)],
                                   acc_sh.at[pl.ds((NS - 1) * SPAN, LAST_SPAN)]))
    pltpu.sync_copy(zdeg, stage1)
    _spans(sid,
           lambda r0: pltpu.sync_copy(stage1, deg_sh.at[pl.ds(r0, SPAN)]),
           lambda: pltpu.sync_copy(stage1.at[pl.ds(0, LAST_SPAN)],
                                   deg_sh.at[pl.ds((NS - 1) * SPAN, LAST_SPAN)]))
    pltpu.sync_copy(srcr.at[wid], idx_s)
    pltpu.sync_copy(dstr.at[wid], idx_d)
    pltpu.sync_copy(ones_in.at[pl.ds(0, CHUNK)], ones_v)
    plsc.subcore_barrier()
    _edge_loop(table, idx_s, idx_d, (r0b, r1b), (g0, g1), (s0, s1), acc_sh,
               deg=(ones_v, deg_sh))
    plsc.subcore_barrier()
    dflat = pl.multiple_of(cid * N, 8)
    _spans(sid,
           lambda r0: (pltpu.sync_copy(acc_sh.at[pl.ds(r0, SPAN)],
                                       acc_out.at[cid].at[pl.ds(r0, SPAN)]),
                       _stage_between(deg_sh, deg_out, stage1,
                                      r0, dflat + r0, SPAN)),
           lambda: (pltpu.sync_copy(
                        acc_sh.at[pl.ds((NS - 1) * SPAN, LAST_SPAN)],
                        acc_out.at[cid].at[pl.ds((NS - 1) * SPAN, LAST_SPAN)]),
                    _stage_between(deg_sh, deg_out, stage1,
                                   (NS - 1) * SPAN, dflat + (NS - 1) * SPAN,
                                   LAST_SPAN)))


def _sc_body(table, srcr, dstr, z64,
             acc_out,
             idx_s, idx_d, r0b, r1b,
             g0, g1, s0, s1, acc_sh):
    cid = lax.axis_index("c")
    sid = lax.axis_index("s")
    wid = cid * NS + sid
    _spans(sid,
           lambda r0: pltpu.sync_copy(z64, acc_sh.at[pl.ds(r0, SPAN)]),
           lambda: pltpu.sync_copy(z64.at[pl.ds(0, LAST_SPAN)],
                                   acc_sh.at[pl.ds((NS - 1) * SPAN, LAST_SPAN)]))
    pltpu.sync_copy(srcr.at[wid], idx_s)
    pltpu.sync_copy(dstr.at[wid], idx_d)
    plsc.subcore_barrier()
    _edge_loop(table, idx_s, idx_d, (r0b, r1b), (g0, g1), (s0, s1), acc_sh)
    plsc.subcore_barrier()
    _spans(sid,
           lambda r0: pltpu.sync_copy(acc_sh.at[pl.ds(r0, SPAN)],
                                      acc_out.at[cid].at[pl.ds(r0, SPAN)]),
           lambda: pltpu.sync_copy(
               acc_sh.at[pl.ds((NS - 1) * SPAN, LAST_SPAN)],
               acc_out.at[cid].at[pl.ds((NS - 1) * SPAN, LAST_SPAN)]))


@functools.lru_cache(maxsize=1)
def _sc_kernels():
    mesh = plsc.VectorSubcoreMesh(core_axis_name="c", subcore_axis_name="s",
                                  num_cores=NC, num_subcores=NS)
    params = pltpu.CompilerParams(use_tc_tiling_on_sc=True)
    segsum_deg = pl.kernel(
        _sc_body_deg,
        out_type=[jax.ShapeDtypeStruct((NC, N, DW), jnp.float32),
                  jax.ShapeDtypeStruct((NC * N,), jnp.float32)],
        mesh=mesh,
        compiler_params=params,
        scratch_types=(
            [pltpu.VMEM((NCHUNK, CHUNK), jnp.int32),
             pltpu.VMEM((NCHUNK, CHUNK), jnp.int32)]
            + [pltpu.VMEM((CHUNK, DW), jnp.float32)] * DEPTH
            + [pltpu.VMEM((CHUNK,), jnp.float32),
               pltpu.VMEM((SPAN,), jnp.float32)]
            + [pltpu.SemaphoreType.DMA] * (2 * DEPTH)
            + [pltpu.VMEM_SHARED((N, DW), jnp.float32),
               pltpu.VMEM_SHARED((N,), jnp.float32)]
        ),
    )
    segsum = pl.kernel(
        _sc_body,
        out_type=jax.ShapeDtypeStruct((NC, N, DW), jnp.float32),
        mesh=mesh,
        compiler_params=params,
        scratch_types=(
            [pltpu.VMEM((NCHUNK, CHUNK), jnp.int32),
             pltpu.VMEM((NCHUNK, CHUNK), jnp.int32)]
            + [pltpu.VMEM((CHUNK, DW), jnp.float32)] * DEPTH
            + [pltpu.SemaphoreType.DMA] * (2 * DEPTH)
            + [pltpu.VMEM_SHARED((N, DW), jnp.float32)]
        ),
    )
    return segsum_deg, segsum


def kernel(x, edge_index, lin0_W, lin0_b, Wn1, Wr1, b1, Wn2, Wr2, b2,
           lin1_W, lin1_b):
    srcr = edge_index[0].reshape(NW, NCHUNK, CHUNK)
    dstr = edge_index[1].reshape(NW, NCHUNK, CHUNK)
    z64 = jnp.zeros((SPAN, DW), jnp.float32)
    zdeg = jnp.zeros((SPAN,), jnp.float32)
    ones_in = jnp.ones((128,), jnp.float32)
    b0r = lin0_b.reshape(1, D_H)
    b1r = b1.reshape(1, D_H)
    b2r = b2.reshape(1, D_H)
    lbr = lin1_b.reshape(1, D_OUT)

    segsum_deg, segsum = _sc_kernels()
    h = _tc_lin0(x, lin0_W, b0r)
    acc1, deg = segsum_deg(h, srcr, dstr, z64, zdeg, ones_in)
    degt = jnp.transpose(deg.reshape(NC, N))
    x1 = _tc_conv(acc1, degt, h, Wn1, Wr1, b1r)
    acc2 = segsum(x1, srcr, dstr, z64)
    return _tc_final(acc2, degt, h, Wn2, Wr2, b2r, lin1_W, lbr)


# speedup vs baseline: 1.2276x; 1.2276x over previous
import functools

import jax
import jax.numpy as jnp
from jax import lax
from jax.experimental import pallas as pl
from jax.experimental.pallas import tpu as pltpu
from jax.experimental.pallas import tpu_sc as plsc

N = 10000
E = 160000
D_IN = 256
D_H = 64
D_OUT = 256

NC = 2
NS = 16
NW = NC * NS
E_PER_W = E // NW
CHUNK = 125
NCHUNK = E_PER_W // CHUNK
SPAN = 640
LAST_SPAN = N - (NS - 1) * SPAN

R = 1000
G = N // R


def _lin0_body(x_ref, w_ref, b_ref, o_ref):
    o = jnp.dot(x_ref[...], w_ref[...], preferred_element_type=jnp.float32)
    o_ref[...] = jnp.maximum(o + b_ref[...], 0.0)


def _tc_lin0(x, w, b):
    return pl.pallas_call(
        _lin0_body,
        grid=(G,),
        in_specs=[
            pl.BlockSpec((R, D_IN), lambda i: (i, 0)),
            pl.BlockSpec((D_IN, D_H), lambda i: (0, 0)),
            pl.BlockSpec((1, D_H), lambda i: (0, 0)),
        ],
        out_specs=pl.BlockSpec((R, D_H), lambda i: (i, 0)),
        out_shape=jax.ShapeDtypeStruct((N, D_H), jnp.float32),
    )(x, w, b)


def _conv_body(acc_ref, degt_ref, h_ref, wn_ref, wr_ref, b_ref, o_ref):
    d = degt_ref[...]
    ds = d[:, 0:1] + d[:, 1:2]
    inv = 1.0 / jnp.maximum(ds, 1.0)
    mean = (acc_ref[0, :, :] + acc_ref[1, :, :]) * inv
    o = (jnp.dot(mean, wn_ref[...], preferred_element_type=jnp.float32)
         + jnp.dot(h_ref[...], wr_ref[...], preferred_element_type=jnp.float32))
    o_ref[...] = jnp.maximum(o + b_ref[...], 0.0)


def _tc_conv(acc, degt, h, wn, wr, b):
    return pl.pallas_call(
        _conv_body,
        grid=(G,),
        in_specs=[
            pl.BlockSpec((2, R, D_H), lambda i: (0, i, 0)),
            pl.BlockSpec((R, 2), lambda i: (i, 0)),
            pl.BlockSpec((R, D_H), lambda i: (i, 0)),
            pl.BlockSpec((D_H, D_H), lambda i: (0, 0)),
            pl.BlockSpec((D_H, D_H), lambda i: (0, 0)),
            pl.BlockSpec((1, D_H), lambda i: (0, 0)),
        ],
        out_specs=pl.BlockSpec((R, D_H), lambda i: (i, 0)),
        out_shape=jax.ShapeDtypeStruct((N, D_H), jnp.float32),
    )(acc, degt, h, wn, wr, b)


def _final_body(acc_ref, degt_ref, x1_ref, wn_ref, wr_ref, b_ref,
                lw_ref, lb_ref, o_ref):
    d = degt_ref[...]
    ds = d[:, 0:1] + d[:, 1:2]
    inv = 1.0 / jnp.maximum(ds, 1.0)
    mean = (acc_ref[0, :, :] + acc_ref[1, :, :]) * inv
    x2 = (jnp.dot(mean, wn_ref[...], preferred_element_type=jnp.float32)
          + jnp.dot(x1_ref[...], wr_ref[...], preferred_element_type=jnp.float32))
    x2 = jnp.maximum(x2 + b_ref[...], 0.0)
    o = jnp.dot(x2, lw_ref[...], preferred_element_type=jnp.float32) + lb_ref[...]
    m = jnp.max(o, axis=1, keepdims=True)
    sh = o - m
    lse = jnp.log(jnp.sum(jnp.exp(sh), axis=1, keepdims=True))
    o_ref[...] = sh - lse


def _tc_final(acc, degt, x1, wn, wr, b, lw, lb):
    return pl.pallas_call(
        _final_body,
        grid=(G,),
        in_specs=[
            pl.BlockSpec((2, R, D_H), lambda i: (0, i, 0)),
            pl.BlockSpec((R, 2), lambda i: (i, 0)),
            pl.BlockSpec((R, D_H), lambda i: (i, 0)),
            pl.BlockSpec((D_H, D_H), lambda i: (0, 0)),
            pl.BlockSpec((D_H, D_H), lambda i: (0, 0)),
            pl.BlockSpec((1, D_H), lambda i: (0, 0)),
            pl.BlockSpec((D_H, D_OUT), lambda i: (0, 0)),
            pl.BlockSpec((1, D_OUT), lambda i: (0, 0)),
        ],
        out_specs=pl.BlockSpec((R, D_OUT), lambda i: (i, 0)),
        out_shape=jax.ShapeDtypeStruct((N, D_OUT), jnp.float32),
    )(acc, degt, x1, wn, wr, b, lw, lb)


def _spans(sid, fn640, fn400):
    @pl.when(sid < NS - 1)
    def _():
        fn640(pl.multiple_of(sid * SPAN, 8))

    @pl.when(sid == NS - 1)
    def _():
        fn400()


def _stage_between(src, dst, stage, src_off, dst_off, length):
    pltpu.sync_copy(src.at[pl.ds(src_off, length)], stage.at[pl.ds(0, length)])
    pltpu.sync_copy(stage.at[pl.ds(0, length)], dst.at[pl.ds(dst_off, length)])


DEPTH = 4
NQ = NCHUNK // DEPTH


def _edge_loop(table, idx_s, idx_d, rows, gsems, ssems, acc_sh, deg=None):
    def gather(c, k):
        pltpu.async_copy(table.at[idx_s.at[c]], rows[k], gsems[k])

    def gwait(k):
        pltpu.make_async_copy(table.at[idx_s.at[0]], rows[k], gsems[k]).wait()

    def swait(k):
        pltpu.make_async_copy(table.at[idx_s.at[0]], rows[k], ssems[k]).wait()

    for k in range(DEPTH):
        gather(k, k)

    def step(i, carry):
        for k in range(DEPTH):
            c = DEPTH * i + k
            gwait(k)
            pltpu.async_copy(rows[k], acc_sh.at[idx_d.at[c]], ssems[k],
                             add=True)
            if deg is not None:
                ones_v, deg_sh = deg
                pltpu.sync_copy(ones_v, deg_sh.at[idx_d.at[c]], add=True)

        @pl.when(i < NQ - 1)
        def _():
            for k in range(DEPTH):
                swait(k)
                gather(DEPTH * (i + 1) + k, k)

        return carry

    lax.fori_loop(0, NQ, step, 0)
    for k in range(DEPTH):
        swait(k)


def _sc_body_deg(table, srcr, dstr, z64, zdeg, ones_in,
                 acc_out, deg_out,
                 idx_s, idx_d, r0b, r1b, r2b, r3b, ones_v, stage, stage1,
                 g0, g1, g2, g3, s0, s1, s2, s3,
                 acc_sh, deg_sh):
    cid = lax.axis_index("c")
    sid = lax.axis_index("s")
    wid = cid * NS + sid
    pltpu.sync_copy(z64, stage)
    pltpu.sync_copy(zdeg, stage1)
    _spans(sid,
           lambda r0: pltpu.sync_copy(stage, acc_sh.at[pl.ds(r0, SPAN)]),
           lambda: pltpu.sync_copy(stage.at[pl.ds(0, LAST_SPAN)],
                                   acc_sh.at[pl.ds((NS - 1) * SPAN, LAST_SPAN)]))
    _spans(sid,
           lambda r0: pltpu.sync_copy(stage1, deg_sh.at[pl.ds(r0, SPAN)]),
           lambda: pltpu.sync_copy(stage1.at[pl.ds(0, LAST_SPAN)],
                                   deg_sh.at[pl.ds((NS - 1) * SPAN, LAST_SPAN)]))
    pltpu.sync_copy(srcr.at[wid], idx_s)
    pltpu.sync_copy(dstr.at[wid], idx_d)
    pltpu.sync_copy(ones_in.at[pl.ds(0, CHUNK)], ones_v)
    plsc.subcore_barrier()
    _edge_loop(table, idx_s, idx_d, (r0b, r1b, r2b, r3b),
               (g0, g1, g2, g3), (s0, s1, s2, s3), acc_sh,
               deg=(ones_v, deg_sh))
    plsc.subcore_barrier()
    dflat = pl.multiple_of(cid * N, 8)
    _spans(sid,
           lambda r0: (_stage_between(acc_sh, acc_out.at[cid], stage, r0, r0, SPAN),
                       _stage_between(deg_sh, deg_out, stage1, r0, dflat + r0, SPAN)),
           lambda: (_stage_between(acc_sh, acc_out.at[cid], stage,
                                   (NS - 1) * SPAN, (NS - 1) * SPAN, LAST_SPAN),
                    _stage_between(deg_sh, deg_out, stage1,
                                   (NS - 1) * SPAN, dflat + (NS - 1) * SPAN,
                                   LAST_SPAN)))


def _sc_body(table, srcr, dstr, z64,
             acc_out,
             idx_s, idx_d, r0b, r1b, r2b, r3b, stage,
             g0, g1, g2, g3, s0, s1, s2, s3, acc_sh):
    cid = lax.axis_index("c")
    sid = lax.axis_index("s")
    wid = cid * NS + sid
    pltpu.sync_copy(z64, stage)
    _spans(sid,
           lambda r0: pltpu.sync_copy(stage, acc_sh.at[pl.ds(r0, SPAN)]),
           lambda: pltpu.sync_copy(stage.at[pl.ds(0, LAST_SPAN)],
                                   acc_sh.at[pl.ds((NS - 1) * SPAN, LAST_SPAN)]))
    pltpu.sync_copy(srcr.at[wid], idx_s)
    pltpu.sync_copy(dstr.at[wid], idx_d)
    plsc.subcore_barrier()
    _edge_loop(table, idx_s, idx_d, (r0b, r1b, r2b, r3b),
               (g0, g1, g2, g3), (s0, s1, s2, s3), acc_sh)
    plsc.subcore_barrier()
    _spans(sid,
           lambda r0: _stage_between(acc_sh, acc_out.at[cid], stage, r0, r0, SPAN),
           lambda: _stage_between(acc_sh, acc_out.at[cid], stage,
                                  (NS - 1) * SPAN, (NS - 1) * SPAN, LAST_SPAN))


@functools.lru_cache(maxsize=1)
def _sc_kernels():
    mesh = plsc.VectorSubcoreMesh(core_axis_name="c", subcore_axis_name="s",
                                  num_cores=NC, num_subcores=NS)
    params = pltpu.CompilerParams(use_tc_tiling_on_sc=False)
    segsum_deg = pl.kernel(
        _sc_body_deg,
        out_type=[jax.ShapeDtypeStruct((NC, N, D_H), jnp.float32),
                  jax.ShapeDtypeStruct((NC * N,), jnp.float32)],
        mesh=mesh,
        compiler_params=params,
        scratch_types=(
            [pltpu.VMEM((NCHUNK, CHUNK), jnp.int32),
             pltpu.VMEM((NCHUNK, CHUNK), jnp.int32)]
            + [pltpu.VMEM((CHUNK, D_H), jnp.float32)] * DEPTH
            + [pltpu.VMEM((CHUNK,), jnp.float32),
               pltpu.VMEM((SPAN, D_H), jnp.float32),
               pltpu.VMEM((SPAN,), jnp.float32)]
            + [pltpu.SemaphoreType.DMA] * (2 * DEPTH)
            + [pltpu.VMEM_SHARED((N, D_H), jnp.float32),
               pltpu.VMEM_SHARED((N,), jnp.float32)]
        ),
    )
    segsum = pl.kernel(
        _sc_body,
        out_type=jax.ShapeDtypeStruct((NC, N, D_H), jnp.float32),
        mesh=mesh,
        compiler_params=params,
        scratch_types=(
            [pltpu.VMEM((NCHUNK, CHUNK), jnp.int32),
             pltpu.VMEM((NCHUNK, CHUNK), jnp.int32)]
            + [pltpu.VMEM((CHUNK, D_H), jnp.float32)] * DEPTH
            + [pltpu.VMEM((SPAN, D_H), jnp.float32)]
            + [pltpu.SemaphoreType.DMA] * (2 * DEPTH)
            + [pltpu.VMEM_SHARED((N, D_H), jnp.float32)]
        ),
    )
    return segsum_deg, segsum


def kernel(x, edge_index, lin0_W, lin0_b, Wn1, Wr1, b1, Wn2, Wr2, b2,
           lin1_W, lin1_b):
    srcr = edge_index[0].reshape(NW, NCHUNK, CHUNK)
    dstr = edge_index[1].reshape(NW, NCHUNK, CHUNK)
    z64 = jnp.zeros((SPAN, D_H), jnp.float32)
    zdeg = jnp.zeros((SPAN,), jnp.float32)
    ones_in = jnp.ones((128,), jnp.float32)
    b0r = lin0_b.reshape(1, D_H)
    b1r = b1.reshape(1, D_H)
    b2r = b2.reshape(1, D_H)
    lbr = lin1_b.reshape(1, D_OUT)

    segsum_deg, segsum = _sc_kernels()
    h = _tc_lin0(x, lin0_W, b0r)
    acc1, deg = segsum_deg(h, srcr, dstr, z64, zdeg, ones_in)
    degt = jnp.transpose(deg.reshape(NC, N))
    x1 = _tc_conv(acc1, degt, h, Wn1, Wr1, b1r)
    acc2 = segsum(x1, srcr, dstr, z64)
    return _tc_final(acc2, degt, h, Wn2, Wr2, b2r, lin1_W, lbr)


# speedup vs baseline: 1.3638x; 1.1110x over previous
import functools

import jax
import jax.numpy as jnp
from jax import lax
from jax.experimental import pallas as pl
from jax.experimental.pallas import tpu as pltpu
from jax.experimental.pallas import tpu_sc as plsc

N = 10000
E = 160000
D_IN = 256
D_H = 64
D_OUT = 256

NC = 2
NS = 16
NW = NC * NS
E_PER_W = E // NW
CH = 128
NFULL = E_PER_W // CH
TAIL_E = E_PER_W - NFULL * CH
NPAIR = (NFULL - 1) // 2
SPAN = 640
LAST_SPAN = N - (NS - 1) * SPAN

R = 2000
G = N // R


def _lin0_body(x_ref, w_ref, b_ref, o_ref):
    o = jnp.dot(x_ref[...], w_ref[...], preferred_element_type=jnp.float32)
    o_ref[...] = jnp.maximum(o + b_ref[...], 0.0)


def _tc_lin0(x, w, b):
    return pl.pallas_call(
        _lin0_body,
        grid=(G,),
        in_specs=[
            pl.BlockSpec((R, D_IN), lambda i: (i, 0)),
            pl.BlockSpec((D_IN, D_H), lambda i: (0, 0)),
            pl.BlockSpec((1, D_H), lambda i: (0, 0)),
        ],
        out_specs=pl.BlockSpec((R, D_H), lambda i: (i, 0)),
        out_shape=jax.ShapeDtypeStruct((N, D_H), jnp.float32),
    )(x, w, b)


def _conv_body(acc_ref, degt_ref, h_ref, wn_ref, wr_ref, b_ref, o_ref):
    d = degt_ref[...]
    ds = d[:, 0:1] + d[:, 1:2]
    inv = 1.0 / jnp.maximum(ds, 1.0)
    mean = (acc_ref[0, :, :] + acc_ref[1, :, :]) * inv
    o = (jnp.dot(mean, wn_ref[...], preferred_element_type=jnp.float32)
         + jnp.dot(h_ref[...], wr_ref[...], preferred_element_type=jnp.float32))
    o_ref[...] = jnp.maximum(o + b_ref[...], 0.0)


def _tc_conv(acc, degt, h, wn, wr, b):
    return pl.pallas_call(
        _conv_body,
        grid=(G,),
        in_specs=[
            pl.BlockSpec((2, R, D_H), lambda i: (0, i, 0)),
            pl.BlockSpec((R, 2), lambda i: (i, 0)),
            pl.BlockSpec((R, D_H), lambda i: (i, 0)),
            pl.BlockSpec((D_H, D_H), lambda i: (0, 0)),
            pl.BlockSpec((D_H, D_H), lambda i: (0, 0)),
            pl.BlockSpec((1, D_H), lambda i: (0, 0)),
        ],
        out_specs=pl.BlockSpec((R, D_H), lambda i: (i, 0)),
        out_shape=jax.ShapeDtypeStruct((N, D_H), jnp.float32),
    )(acc, degt, h, wn, wr, b)


def _final_body(acc_ref, degt_ref, x1_ref, wn_ref, wr_ref, b_ref,
                lw_ref, lb_ref, o_ref):
    d = degt_ref[...]
    ds = d[:, 0:1] + d[:, 1:2]
    inv = 1.0 / jnp.maximum(ds, 1.0)
    mean = (acc_ref[0, :, :] + acc_ref[1, :, :]) * inv
    x2 = (jnp.dot(mean, wn_ref[...], preferred_element_type=jnp.float32)
          + jnp.dot(x1_ref[...], wr_ref[...], preferred_element_type=jnp.float32))
    x2 = jnp.maximum(x2 + b_ref[...], 0.0)
    o = jnp.dot(x2, lw_ref[...], preferred_element_type=jnp.float32) + lb_ref[...]
    m = jnp.max(o, axis=1, keepdims=True)
    sh = o - m
    lse = jnp.log(jnp.sum(jnp.exp(sh), axis=1, keepdims=True))
    o_ref[...] = sh - lse


def _tc_final(acc, degt, x1, wn, wr, b, lw, lb):
    return pl.pallas_call(
        _final_body,
        grid=(G,),
        in_specs=[
            pl.BlockSpec((2, R, D_H), lambda i: (0, i, 0)),
            pl.BlockSpec((R, 2), lambda i: (i, 0)),
            pl.BlockSpec((R, D_H), lambda i: (i, 0)),
            pl.BlockSpec((D_H, D_H), lambda i: (0, 0)),
            pl.BlockSpec((D_H, D_H), lambda i: (0, 0)),
            pl.BlockSpec((1, D_H), lambda i: (0, 0)),
            pl.BlockSpec((D_H, D_OUT), lambda i: (0, 0)),
            pl.BlockSpec((1, D_OUT), lambda i: (0, 0)),
        ],
        out_specs=pl.BlockSpec((R, D_OUT), lambda i: (i, 0)),
        out_shape=jax.ShapeDtypeStruct((N, D_OUT), jnp.float32),
    )(acc, degt, x1, wn, wr, b, lw, lb)


def _spans(sid, fn640, fn400):
    @pl.when(sid < NS - 1)
    def _():
        fn640(pl.multiple_of(sid * SPAN, 8))

    @pl.when(sid == NS - 1)
    def _():
        fn400()


def _stage_between(src, dst, stage, src_off, dst_off, length):
    pltpu.sync_copy(src.at[pl.ds(src_off, length)], stage.at[pl.ds(0, length)])
    pltpu.sync_copy(stage.at[pl.ds(0, length)], dst.at[pl.ds(dst_off, length)])


def _edge_loop(table, idx_s, idx_d, rows, gsems, ssems, acc_sh, deg=None):
    def cref(ref, c, ln=CH):
        return ref.at[pl.ds(pl.multiple_of(c * CH, 8), ln)]

    def gather(c, k):
        pltpu.async_copy(table.at[cref(idx_s, c)], rows[k], gsems[k])

    def gwait(k):
        pltpu.make_async_copy(table.at[cref(idx_s, 0)], rows[k],
                              gsems[k]).wait()

    def swait(k):
        pltpu.make_async_copy(table.at[cref(idx_s, 0)], rows[k],
                              ssems[k]).wait()

    def scat(c, k):
        pltpu.async_copy(rows[k], acc_sh.at[cref(idx_d, c)], ssems[k],
                         add=True)
        if deg is not None:
            ones_v, deg_sh = deg
            pltpu.sync_copy(ones_v, deg_sh.at[cref(idx_d, c)], add=True)

    gather(0, 0)
    gather(1, 1)

    def step(i, carry):
        gwait(0)
        scat(2 * i, 0)

        @pl.when(i < NPAIR - 1)
        def _():
            swait(0)
            gather(2 * i + 2, 0)

        gwait(1)
        scat(2 * i + 1, 1)

        @pl.when(i < NPAIR - 1)
        def _():
            swait(1)
            gather(2 * i + 3, 1)

        return carry

    lax.fori_loop(0, NPAIR, step, 0)
    swait(0)
    swait(1)
    gather(NFULL - 1, 0)
    t0 = NFULL * CH
    pltpu.async_copy(table.at[idx_s.at[pl.ds(t0, TAIL_E)]],
                     rows[1].at[pl.ds(0, TAIL_E)], gsems[1])
    gwait(0)
    scat(NFULL - 1, 0)
    pltpu.make_async_copy(table.at[idx_s.at[pl.ds(t0, TAIL_E)]],
                          rows[1].at[pl.ds(0, TAIL_E)], gsems[1]).wait()
    pltpu.async_copy(rows[1].at[pl.ds(0, TAIL_E)],
                     acc_sh.at[idx_d.at[pl.ds(t0, TAIL_E)]], ssems[1],
                     add=True)
    if deg is not None:
        ones_v, deg_sh = deg
        pltpu.sync_copy(ones_v.at[pl.ds(0, TAIL_E)],
                        deg_sh.at[idx_d.at[pl.ds(t0, TAIL_E)]], add=True)
    swait(0)
    pltpu.make_async_copy(table.at[idx_s.at[pl.ds(t0, TAIL_E)]],
                          rows[1].at[pl.ds(0, TAIL_E)], ssems[1]).wait()


def _sc_body_deg(table, edges, z64, zdeg, ones_in,
                 acc_out, deg_out,
                 idx_s, idx_d, r0b, r1b, ones_v, stage, stage1,
                 g0, g1, s0, s1,
                 acc_sh, deg_sh):
    cid = lax.axis_index("c")
    sid = lax.axis_index("s")
    wid = cid * NS + sid
    pltpu.sync_copy(z64, stage)
    pltpu.sync_copy(zdeg, stage1)
    _spans(sid,
           lambda r0: pltpu.sync_copy(stage, acc_sh.at[pl.ds(r0, SPAN)]),
           lambda: pltpu.sync_copy(stage.at[pl.ds(0, LAST_SPAN)],
                                   acc_sh.at[pl.ds((NS - 1) * SPAN, LAST_SPAN)]))
    _spans(sid,
           lambda r0: pltpu.sync_copy(stage1, deg_sh.at[pl.ds(r0, SPAN)]),
           lambda: pltpu.sync_copy(stage1.at[pl.ds(0, LAST_SPAN)],
                                   deg_sh.at[pl.ds((NS - 1) * SPAN, LAST_SPAN)]))
    ebase = pl.multiple_of(wid * E_PER_W, 8)
    pltpu.sync_copy(edges.at[0].at[pl.ds(ebase, E_PER_W)], idx_s)
    pltpu.sync_copy(edges.at[1].at[pl.ds(ebase, E_PER_W)], idx_d)
    pltpu.sync_copy(ones_in, ones_v)
    plsc.subcore_barrier()
    _edge_loop(table, idx_s, idx_d, (r0b, r1b),
               (g0, g1), (s0, s1), acc_sh,
               deg=(ones_v, deg_sh))
    plsc.subcore_barrier()
    dflat = pl.multiple_of(cid * N, 8)
    _spans(sid,
           lambda r0: (_stage_between(acc_sh, acc_out.at[cid], stage, r0, r0, SPAN),
                       _stage_between(deg_sh, deg_out, stage1, r0, dflat + r0, SPAN)),
           lambda: (_stage_between(acc_sh, acc_out.at[cid], stage,
                                   (NS - 1) * SPAN, (NS - 1) * SPAN, LAST_SPAN),
                    _stage_between(deg_sh, deg_out, stage1,
                                   (NS - 1) * SPAN, dflat + (NS - 1) * SPAN,
                                   LAST_SPAN)))


def _sc_body(table, edges, z64,
             acc_out,
             idx_s, idx_d, r0b, r1b, stage,
             g0, g1, s0, s1, acc_sh):
    cid = lax.axis_index("c")
    sid = lax.axis_index("s")
    wid = cid * NS + sid
    pltpu.sync_copy(z64, stage)
    _spans(sid,
           lambda r0: pltpu.sync_copy(stage, acc_sh.at[pl.ds(r0, SPAN)]),
           lambda: pltpu.sync_copy(stage.at[pl.ds(0, LAST_SPAN)],
                                   acc_sh.at[pl.ds((NS - 1) * SPAN, LAST_SPAN)]))
    ebase = pl.multiple_of(wid * E_PER_W, 8)
    pltpu.sync_copy(edges.at[0].at[pl.ds(ebase, E_PER_W)], idx_s)
    pltpu.sync_copy(edges.at[1].at[pl.ds(ebase, E_PER_W)], idx_d)
    plsc.subcore_barrier()
    _edge_loop(table, idx_s, idx_d, (r0b, r1b),
               (g0, g1), (s0, s1), acc_sh)
    plsc.subcore_barrier()
    _spans(sid,
           lambda r0: _stage_between(acc_sh, acc_out.at[cid], stage, r0, r0, SPAN),
           lambda: _stage_between(acc_sh, acc_out.at[cid], stage,
                                  (NS - 1) * SPAN, (NS - 1) * SPAN, LAST_SPAN))


@functools.lru_cache(maxsize=1)
def _sc_kernels():
    mesh = plsc.VectorSubcoreMesh(core_axis_name="c", subcore_axis_name="s",
                                  num_cores=NC, num_subcores=NS)
    params = pltpu.CompilerParams(use_tc_tiling_on_sc=False)
    segsum_deg = pl.kernel(
        _sc_body_deg,
        out_type=[jax.ShapeDtypeStruct((NC, N, D_H), jnp.float32),
                  jax.ShapeDtypeStruct((NC * N,), jnp.float32)],
        mesh=mesh,
        compiler_params=params,
        scratch_types=(
            [pltpu.VMEM((E_PER_W,), jnp.int32),
             pltpu.VMEM((E_PER_W,), jnp.int32)]
            + [pltpu.VMEM((CH, D_H), jnp.float32)] * 2
            + [pltpu.VMEM((CH,), jnp.float32),
               pltpu.VMEM((SPAN, D_H), jnp.float32),
               pltpu.VMEM((SPAN,), jnp.float32)]
            + [pltpu.SemaphoreType.DMA] * 4
            + [pltpu.VMEM_SHARED((N, D_H), jnp.float32),
               pltpu.VMEM_SHARED((N,), jnp.float32)]
        ),
    )
    segsum = pl.kernel(
        _sc_body,
        out_type=jax.ShapeDtypeStruct((NC, N, D_H), jnp.float32),
        mesh=mesh,
        compiler_params=params,
        scratch_types=(
            [pltpu.VMEM((E_PER_W,), jnp.int32),
             pltpu.VMEM((E_PER_W,), jnp.int32)]
            + [pltpu.VMEM((CH, D_H), jnp.float32)] * 2
            + [pltpu.VMEM((SPAN, D_H), jnp.float32)]
            + [pltpu.SemaphoreType.DMA] * 4
            + [pltpu.VMEM_SHARED((N, D_H), jnp.float32)]
        ),
    )
    return segsum_deg, segsum


def kernel(x, edge_index, lin0_W, lin0_b, Wn1, Wr1, b1, Wn2, Wr2, b2,
           lin1_W, lin1_b):
    z64 = jnp.zeros((SPAN, D_H), jnp.float32)
    zdeg = jnp.zeros((SPAN,), jnp.float32)
    ones_in = jnp.ones((128,), jnp.float32)
    b0r = lin0_b.reshape(1, D_H)
    b1r = b1.reshape(1, D_H)
    b2r = b2.reshape(1, D_H)
    lbr = lin1_b.reshape(1, D_OUT)

    segsum_deg, segsum = _sc_kernels()
    h = _tc_lin0(x, lin0_W, b0r)
    acc1, deg = segsum_deg(h, edge_index, z64, zdeg, ones_in)
    degt = jnp.transpose(deg.reshape(NC, N))
    x1 = _tc_conv(acc1, degt, h, Wn1, Wr1, b1r)
    acc2 = segsum(x1, edge_index, z64)
    return _tc_final(acc2, degt, h, Wn2, Wr2, b2r, lin1_W, lbr)


# speedup vs baseline: 1.4012x; 1.0274x over previous
import functools

import jax
import jax.numpy as jnp
from jax import lax
from jax.experimental import pallas as pl
from jax.experimental.pallas import tpu as pltpu
from jax.experimental.pallas import tpu_sc as plsc

N = 10000
E = 160000
D_IN = 256
D_H = 64
D_OUT = 256

NC = 2
NS = 16
NW = NC * NS
E_PER_W = E // NW
CH = 128
NFULL = E_PER_W // CH
TAIL_E = E_PER_W - NFULL * CH
NPAIR = (NFULL - 1) // 2
SPAN = 640
LAST_SPAN = N - (NS - 1) * SPAN

R = 2000
G = N // R


def _lin0_body(x_ref, w_ref, b_ref, o_ref):
    o = jnp.dot(x_ref[...], w_ref[...], preferred_element_type=jnp.float32)
    o_ref[...] = jnp.maximum(o + b_ref[...], 0.0)


def _tc_lin0(x, w, b):
    return pl.pallas_call(
        _lin0_body,
        grid=(G,),
        in_specs=[
            pl.BlockSpec((R, D_IN), lambda i: (i, 0)),
            pl.BlockSpec((D_IN, D_H), lambda i: (0, 0)),
            pl.BlockSpec((1, D_H), lambda i: (0, 0)),
        ],
        out_specs=pl.BlockSpec((R, D_H), lambda i: (i, 0)),
        out_shape=jax.ShapeDtypeStruct((N, D_H), jnp.float32),
    )(x, w, b)


def _conv_body(acc_ref, degt_ref, h_ref, wn_ref, wr_ref, b_ref, o_ref):
    d = degt_ref[...]
    ds = d[:, 0:1] + d[:, 1:2]
    inv = 1.0 / jnp.maximum(ds, 1.0)
    mean = (acc_ref[0, :, :] + acc_ref[1, :, :]) * inv
    o = (jnp.dot(mean, wn_ref[...], preferred_element_type=jnp.float32)
         + jnp.dot(h_ref[...], wr_ref[...], preferred_element_type=jnp.float32))
    o_ref[...] = jnp.maximum(o + b_ref[...], 0.0)


def _tc_conv(acc, degt, h, wn, wr, b):
    return pl.pallas_call(
        _conv_body,
        grid=(G,),
        in_specs=[
            pl.BlockSpec((2, R, D_H), lambda i: (0, i, 0)),
            pl.BlockSpec((R, 2), lambda i: (i, 0)),
            pl.BlockSpec((R, D_H), lambda i: (i, 0)),
            pl.BlockSpec((D_H, D_H), lambda i: (0, 0)),
            pl.BlockSpec((D_H, D_H), lambda i: (0, 0)),
            pl.BlockSpec((1, D_H), lambda i: (0, 0)),
        ],
        out_specs=pl.BlockSpec((R, D_H), lambda i: (i, 0)),
        out_shape=jax.ShapeDtypeStruct((N, D_H), jnp.float32),
    )(acc, degt, h, wn, wr, b)


def _final_body(acc_ref, degt_ref, x1_ref, wn_ref, wr_ref, b_ref,
                lw_ref, lb_ref, o_ref):
    d = degt_ref[...]
    ds = d[:, 0:1] + d[:, 1:2]
    inv = 1.0 / jnp.maximum(ds, 1.0)
    mean = (acc_ref[0, :, :] + acc_ref[1, :, :]) * inv
    x2 = (jnp.dot(mean, wn_ref[...], preferred_element_type=jnp.float32)
          + jnp.dot(x1_ref[...], wr_ref[...], preferred_element_type=jnp.float32))
    x2 = jnp.maximum(x2 + b_ref[...], 0.0)
    o = jnp.dot(x2, lw_ref[...], preferred_element_type=jnp.float32) + lb_ref[...]
    m = jnp.max(o, axis=1, keepdims=True)
    sh = o - m
    lse = jnp.log(jnp.sum(jnp.exp(sh), axis=1, keepdims=True))
    o_ref[...] = sh - lse


def _tc_final(acc, degt, x1, wn, wr, b, lw, lb):
    return pl.pallas_call(
        _final_body,
        grid=(G,),
        in_specs=[
            pl.BlockSpec((2, R, D_H), lambda i: (0, i, 0)),
            pl.BlockSpec((R, 2), lambda i: (i, 0)),
            pl.BlockSpec((R, D_H), lambda i: (i, 0)),
            pl.BlockSpec((D_H, D_H), lambda i: (0, 0)),
            pl.BlockSpec((D_H, D_H), lambda i: (0, 0)),
            pl.BlockSpec((1, D_H), lambda i: (0, 0)),
            pl.BlockSpec((D_H, D_OUT), lambda i: (0, 0)),
            pl.BlockSpec((1, D_OUT), lambda i: (0, 0)),
        ],
        out_specs=pl.BlockSpec((R, D_OUT), lambda i: (i, 0)),
        out_shape=jax.ShapeDtypeStruct((N, D_OUT), jnp.float32),
    )(acc, degt, x1, wn, wr, b, lw, lb)


def _spans(sid, fn640, fn400):
    @pl.when(sid < NS - 1)
    def _():
        fn640(pl.multiple_of(sid * SPAN, 8))

    @pl.when(sid == NS - 1)
    def _():
        fn400()


def _stage_between(src, dst, stage, src_off, dst_off, length):
    pltpu.sync_copy(src.at[pl.ds(src_off, length)], stage.at[pl.ds(0, length)])
    pltpu.sync_copy(stage.at[pl.ds(0, length)], dst.at[pl.ds(dst_off, length)])


def _edge_loop(table, idx_s, idx_d, rows, gsems, ssems, acc_sh, deg=None):
    def cref(ref, c, ln=CH):
        return ref.at[pl.ds(pl.multiple_of(c * CH, 8), ln)]

    def gather(c, k):
        pltpu.async_copy(table.at[cref(idx_s, c)], rows[k], gsems[k])

    def gwait(k):
        pltpu.make_async_copy(table.at[cref(idx_s, 0)], rows[k],
                              gsems[k]).wait()

    def swait(k):
        pltpu.make_async_copy(table.at[cref(idx_s, 0)], rows[k],
                              ssems[k]).wait()

    def scat(c, k):
        pltpu.async_copy(rows[k], acc_sh.at[cref(idx_d, c)], ssems[k],
                         add=True)
        if deg is not None:
            ones_v, deg_sh = deg
            pltpu.sync_copy(ones_v, deg_sh.at[cref(idx_d, c)], add=True)

    gather(0, 0)
    gather(1, 1)

    def step(i, carry):
        gwait(0)
        scat(2 * i, 0)

        @pl.when(i < NPAIR - 1)
        def _():
            swait(0)
            gather(2 * i + 2, 0)

        gwait(1)
        scat(2 * i + 1, 1)

        @pl.when(i < NPAIR - 1)
        def _():
            swait(1)
            gather(2 * i + 3, 1)

        return carry

    lax.fori_loop(0, NPAIR, step, 0)
    swait(0)
    swait(1)
    gather(NFULL - 1, 0)
    t0 = NFULL * CH
    pltpu.async_copy(table.at[idx_s.at[pl.ds(t0, TAIL_E)]],
                     rows[1].at[pl.ds(0, TAIL_E)], gsems[1])
    gwait(0)
    scat(NFULL - 1, 0)
    pltpu.make_async_copy(table.at[idx_s.at[pl.ds(t0, TAIL_E)]],
                          rows[1].at[pl.ds(0, TAIL_E)], gsems[1]).wait()
    pltpu.async_copy(rows[1].at[pl.ds(0, TAIL_E)],
                     acc_sh.at[idx_d.at[pl.ds(t0, TAIL_E)]], ssems[1],
                     add=True)
    if deg is not None:
        ones_v, deg_sh = deg
        pltpu.sync_copy(ones_v.at[pl.ds(0, TAIL_E)],
                        deg_sh.at[idx_d.at[pl.ds(t0, TAIL_E)]], add=True)
    swait(0)
    pltpu.make_async_copy(table.at[idx_s.at[pl.ds(t0, TAIL_E)]],
                          rows[1].at[pl.ds(0, TAIL_E)], ssems[1]).wait()


def _sc_body_deg(table, edges, z64, zdeg, ones_in,
                 acc_out, deg_out,
                 idx_s, idx_d, r0b, r1b, ones_v, stage1,
                 g0, g1, s0, s1,
                 acc_sh, deg_sh):
    cid = lax.axis_index("c")
    sid = lax.axis_index("s")
    wid = cid * NS + sid
    pltpu.sync_copy(zdeg, stage1)
    _spans(sid,
           lambda r0: pltpu.sync_copy(z64, acc_sh.at[pl.ds(r0, SPAN)]),
           lambda: pltpu.sync_copy(z64.at[pl.ds(0, LAST_SPAN)],
                                   acc_sh.at[pl.ds((NS - 1) * SPAN, LAST_SPAN)]))
    _spans(sid,
           lambda r0: pltpu.sync_copy(stage1, deg_sh.at[pl.ds(r0, SPAN)]),
           lambda: pltpu.sync_copy(stage1.at[pl.ds(0, LAST_SPAN)],
                                   deg_sh.at[pl.ds((NS - 1) * SPAN, LAST_SPAN)]))
    ebase = pl.multiple_of(wid * E_PER_W, 8)
    pltpu.sync_copy(edges.at[0].at[pl.ds(ebase, E_PER_W)], idx_s)
    pltpu.sync_copy(edges.at[1].at[pl.ds(ebase, E_PER_W)], idx_d)
    pltpu.sync_copy(ones_in, ones_v)
    plsc.subcore_barrier()
    _edge_loop(table, idx_s, idx_d, (r0b, r1b),
               (g0, g1), (s0, s1), acc_sh,
               deg=(ones_v, deg_sh))
    plsc.subcore_barrier()
    dflat = pl.multiple_of(cid * N, 8)
    _spans(sid,
           lambda r0: (pltpu.sync_copy(acc_sh.at[pl.ds(r0, SPAN)],
                                       acc_out.at[cid].at[pl.ds(r0, SPAN)]),
                       _stage_between(deg_sh, deg_out, stage1, r0, dflat + r0, SPAN)),
           lambda: (pltpu.sync_copy(
                        acc_sh.at[pl.ds((NS - 1) * SPAN, LAST_SPAN)],
                        acc_out.at[cid].at[pl.ds((NS - 1) * SPAN, LAST_SPAN)]),
                    _stage_between(deg_sh, deg_out, stage1,
                                   (NS - 1) * SPAN, dflat + (NS - 1) * SPAN,
                                   LAST_SPAN)))


def _sc_body(table, edges, z64,
             acc_out,
             idx_s, idx_d, r0b, r1b,
             g0, g1, s0, s1, acc_sh):
    cid = lax.axis_index("c")
    sid = lax.axis_index("s")
    wid = cid * NS + sid
    _spans(sid,
           lambda r0: pltpu.sync_copy(z64, acc_sh.at[pl.ds(r0, SPAN)]),
           lambda: pltpu.sync_copy(z64.at[pl.ds(0, LAST_SPAN)],
                                   acc_sh.at[pl.ds((NS - 1) * SPAN, LAST_SPAN)]))
    ebase = pl.multiple_of(wid * E_PER_W, 8)
    pltpu.sync_copy(edges.at[0].at[pl.ds(ebase, E_PER_W)], idx_s)
    pltpu.sync_copy(edges.at[1].at[pl.ds(ebase, E_PER_W)], idx_d)
    plsc.subcore_barrier()
    _edge_loop(table, idx_s, idx_d, (r0b, r1b),
               (g0, g1), (s0, s1), acc_sh)
    plsc.subcore_barrier()
    _spans(sid,
           lambda r0: pltpu.sync_copy(acc_sh.at[pl.ds(r0, SPAN)],
                                      acc_out.at[cid].at[pl.ds(r0, SPAN)]),
           lambda: pltpu.sync_copy(
               acc_sh.at[pl.ds((NS - 1) * SPAN, LAST_SPAN)],
               acc_out.at[cid].at[pl.ds((NS - 1) * SPAN, LAST_SPAN)]))


@functools.lru_cache(maxsize=1)
def _sc_kernels():
    mesh = plsc.VectorSubcoreMesh(core_axis_name="c", subcore_axis_name="s",
                                  num_cores=NC, num_subcores=NS)
    params = pltpu.CompilerParams(use_tc_tiling_on_sc=False)
    segsum_deg = pl.kernel(
        _sc_body_deg,
        out_type=[jax.ShapeDtypeStruct((NC, N, D_H), jnp.float32),
                  jax.ShapeDtypeStruct((NC * N,), jnp.float32)],
        mesh=mesh,
        compiler_params=params,
        scratch_types=(
            [pltpu.VMEM((E_PER_W,), jnp.int32),
             pltpu.VMEM((E_PER_W,), jnp.int32)]
            + [pltpu.VMEM((CH, D_H), jnp.float32)] * 2
            + [pltpu.VMEM((CH,), jnp.float32),
               pltpu.VMEM((SPAN,), jnp.float32)]
            + [pltpu.SemaphoreType.DMA] * 4
            + [pltpu.VMEM_SHARED((N, D_H), jnp.float32),
               pltpu.VMEM_SHARED((N,), jnp.float32)]
        ),
    )
    segsum = pl.kernel(
        _sc_body,
        out_type=jax.ShapeDtypeStruct((NC, N, D_H), jnp.float32),
        mesh=mesh,
        compiler_params=params,
        scratch_types=(
            [pltpu.VMEM((E_PER_W,), jnp.int32),
             pltpu.VMEM((E_PER_W,), jnp.int32)]
            + [pltpu.VMEM((CH, D_H), jnp.float32)] * 2
            + [pltpu.SemaphoreType.DMA] * 4
            + [pltpu.VMEM_SHARED((N, D_H), jnp.float32)]
        ),
    )
    return segsum_deg, segsum


def kernel(x, edge_index, lin0_W, lin0_b, Wn1, Wr1, b1, Wn2, Wr2, b2,
           lin1_W, lin1_b):
    z64 = jnp.zeros((SPAN, D_H), jnp.float32)
    zdeg = jnp.zeros((SPAN,), jnp.float32)
    ones_in = jnp.ones((128,), jnp.float32)
    b0r = lin0_b.reshape(1, D_H)
    b1r = b1.reshape(1, D_H)
    b2r = b2.reshape(1, D_H)
    lbr = lin1_b.reshape(1, D_OUT)

    segsum_deg, segsum = _sc_kernels()
    h = _tc_lin0(x, lin0_W, b0r)
    acc1, deg = segsum_deg(h, edge_index, z64, zdeg, ones_in)
    degt = jnp.transpose(deg.reshape(NC, N))
    x1 = _tc_conv(acc1, degt, h, Wn1, Wr1, b1r)
    acc2 = segsum(x1, edge_index, z64)
    return _tc_final(acc2, degt, h, Wn2, Wr2, b2r, lin1_W, lbr)
